# baseline (speedup 1.0000x reference)
import functools

import jax
import jax.numpy as jnp
from jax import lax
from jax.experimental import pallas as pl
from jax.experimental.pallas import tpu as pltpu
from jax.experimental.pallas import tpu_sc as plsc

B = 16384
L = 200
E = 32
H = 64
O = 2

NC = 2
NS = 16
NW = NC * NS
RPW = B // NW
CB = 128
NCHUNK = RPW // CB
LAG = 32


def _sc_pool_sum(text, emb_table):
    mesh = plsc.VectorSubcoreMesh(core_axis_name="c", subcore_axis_name="s")

    @functools.partial(
        pl.kernel,
        mesh=mesh,
        out_type=jax.ShapeDtypeStruct((B, E), jnp.float32),
        scratch_types=[
            pltpu.VMEM((CB, L), jnp.int32),
            pltpu.VMEM((L, CB), jnp.int32),
            pltpu.VMEM((CB, E), jnp.float32),
            pltpu.SemaphoreType.DMA,
        ],
        compiler_params=pltpu.CompilerParams(
            use_tc_tiling_on_sc=False, needs_layout_passes=False),
    )
    def k(text_hbm, table_hbm, out_hbm, txt_v, idx_v, acc_v, gsem):
        cid = lax.axis_index("c")
        sid = lax.axis_index("s")
        wid = sid * NC + cid
        base = wid * RPW

        lane = lax.iota(jnp.int32, 16)
        zero = jnp.zeros((16,), jnp.float32)

        def issue(t):
            return pltpu.async_copy(
                table_hbm.at[idx_v.at[t]], acc_v, gsem, add=True)

        def drain(t):
            pltpu.make_async_copy(
                table_hbm.at[idx_v.at[t]], acc_v, gsem).wait()

        def chunk_body(ci, _):
            cbase = base + ci * CB

            pltpu.sync_copy(text_hbm.at[pl.ds(cbase, CB), :], txt_v)

            def zbody(i, _):
                acc_v[i, pl.ds(0, 16)] = zero
                acc_v[i, pl.ds(16, 16)] = zero
                return 0

            lax.fori_loop(0, CB, zbody, 0)

            def gbody(t, _):
                tcol = jnp.full((16,), t, jnp.int32)
                for r0 in range(CB // 16):
                    rows = lane + (r0 * 16)
                    vals = plsc.load_gather(txt_v, [rows, tcol])
                    idx_v[t, pl.ds(r0 * 16, 16)] = vals
                issue(t)

                @pl.when(t >= LAG)
                def _():
                    drain(t - LAG)

                return 0

            lax.fori_loop(0, L, gbody, 0)
            for tt in range(L - LAG, L):
                drain(tt)

            pltpu.sync_copy(acc_v, out_hbm.at[pl.ds(cbase, CB)])
            return 0

        lax.fori_loop(0, NCHUNK, chunk_body, 0)

    return k(text, emb_table)


def _mlp_kernel(x_ref, w1_ref, b1_ref, w2_ref, b2_ref, o_ref):
    x = x_ref[...] * (1.0 / L)
    h = jnp.dot(x, w1_ref[...], preferred_element_type=jnp.float32)
    h = jnp.maximum(h + b1_ref[...], 0.0)
    o = jnp.dot(h, w2_ref[...], preferred_element_type=jnp.float32)
    o = o + b2_ref[...]
    o_ref[...] = 1.0 / (1.0 + jnp.exp(-o))


def kernel(text, emb_table, W1, b1, W2, b2):
    pooled_sum = _sc_pool_sum(text.astype(jnp.int32), emb_table)

    bt = 2048
    out = pl.pallas_call(
        _mlp_kernel,
        out_shape=jax.ShapeDtypeStruct((B, O), jnp.float32),
        grid=(B // bt,),
        in_specs=[
            pl.BlockSpec((bt, E), lambda i: (i, 0)),
            pl.BlockSpec((E, H), lambda i: (0, 0)),
            pl.BlockSpec((1, H), lambda i: (0, 0)),
            pl.BlockSpec((H, O), lambda i: (0, 0)),
            pl.BlockSpec((1, O), lambda i: (0, 0)),
        ],
        out_specs=pl.BlockSpec((bt, O), lambda i: (i, 0)),
    )(pooled_sum, W1.T, b1[None, :], W2.T, b2[None, :])
    return out
